# Initial kernel scaffold; baseline (speedup 1.0000x reference)
import jax
import jax.numpy as jnp
from jax.experimental import pallas as pl


def kernel(x, edge_index):
    raise NotImplementedError("write your pallas kernel here")


# speedup vs baseline: 59.3086x; 59.3086x over previous
import functools
import numpy as np
import jax
import jax.numpy as jnp
from jax import lax
from jax._src import config as _jax_config
from jax.experimental import pallas as pl
from jax.experimental.pallas import tpu as pltpu
from jax.experimental.pallas import tpu_sc as plsc

_N = 10000
_D = 128
_E = 320000

_LANE = 128
_N_PAD = 10240
_ROW_BLK = 256
_JB = _N_PAD // _LANE

_NUM_TILES = 32
_CHUNK = 128
_E_TILE = 10240
_NCHUNK = _E_TILE // _CHUNK
_E_PAD = _NUM_TILES * _E_TILE
_TRASH = _N
_GW = 16

_rng_k = np.random.default_rng(42)
_R64 = _rng_k.integers(0, 2**64, size=(_D,), dtype=np.uint64) | np.uint64(1)
_R_LO = (_R64 & np.uint64(0xFFFFFFFF)).astype(np.uint32).reshape(1, _D)
_R_HI = (_R64 >> np.uint64(32)).astype(np.uint32).reshape(1, _D)

_M16 = np.uint32(0xFFFF)
_M12 = np.uint32(0xFFF)

_C1_LO, _C1_HI = np.uint32(0x7F4A7C15), np.uint32(0x9E3779B9)
_C2_LO, _C2_HI = np.uint32(0x1CE4E5B9), np.uint32(0xBF58476D)
_C3_LO, _C3_HI = np.uint32(0x133111EB), np.uint32(0x94D049BB)
_CX_LO, _CX_HI = np.uint32(0x6659FD93), np.uint32(0xD6E8FEB8)
_CK_LO, _CK_HI = np.uint32(0x000001B3), np.uint32(0x00000100)


def _mulhi32(a, b):
    al = a & _M16
    ah = a >> 16
    bl = b & _M16
    bh = b >> 16
    ll = al * bl
    lh = al * bh
    hl = ah * bl
    hh = ah * bh
    mid = lh + (ll >> 16)
    mid2 = hl + (mid & _M16)
    return hh + (mid >> 16) + (mid2 >> 16)


def _mul64(alo, ahi, blo, bhi):
    lo = alo * blo
    hi = _mulhi32(alo, blo) + alo * bhi + ahi * blo
    return lo, hi


def _add64(alo, ahi, blo, bhi):
    lo = alo + blo
    carry = (lo < blo).astype(jnp.uint32)
    return lo, ahi + bhi + carry


def _shr64(lo, hi, k):
    return (lo >> k) | (hi << (32 - k)), hi >> k


def _mix64(lo, hi):
    lo, hi = _add64(lo, hi, _C1_LO, _C1_HI)
    slo, shi = _shr64(lo, hi, 30)
    lo, hi = lo ^ slo, hi ^ shi
    lo, hi = _mul64(lo, hi, _C2_LO, _C2_HI)
    slo, shi = _shr64(lo, hi, 27)
    lo, hi = lo ^ slo, hi ^ shi
    lo, hi = _mul64(lo, hi, _C3_LO, _C3_HI)
    slo, shi = _shr64(lo, hi, 31)
    return lo ^ slo, hi ^ shi


def _hash_body(xb_ref, rlo_ref, rhi_ref, hlo_ref, hhi_ref,
               l0, l1, l2, l3, l4, l5):
    b = xb_ref[...]
    rlo = rlo_ref[...]
    rhi = rhi_ref[...]
    tlo = b * rlo
    thi = _mulhi32(b, rlo) + b * rhi
    def _usum(v):
        s = jnp.sum(lax.bitcast_convert_type(v, jnp.int32), axis=1,
                    dtype=jnp.int32)
        return lax.bitcast_convert_type(s, jnp.uint32)

    sl = _usum(tlo & _M16)
    sh = _usum(tlo >> 16)
    acc_lo = sl + (sh << 16)
    carry = (sh + (sl >> 16)) >> 16
    acc_hi = _usum(thi) + carry
    hlo, hhi = _mix64(acc_lo, acc_hi)
    hlo_ref[...] = hlo
    hhi_ref[...] = hhi
    glo, ghi = _mix64(hlo ^ _CX_LO, hhi ^ _CX_HI)
    l0[...] = (glo & _M12).astype(jnp.int32)
    l1[...] = ((glo >> 12) & _M12).astype(jnp.int32)
    l2[...] = (((glo >> 24) | (ghi << 8)) & _M12).astype(jnp.int32)
    l3[...] = ((ghi >> 4) & _M12).astype(jnp.int32)
    l4[...] = ((ghi >> 16) & _M12).astype(jnp.int32)
    l5[...] = (ghi >> 28).astype(jnp.int32)


def _run_hash(x_bits_pad):
    grid = (_N_PAD // _ROW_BLK,)
    vec_spec = pl.BlockSpec((_ROW_BLK,), lambda i: (i,))
    outs = [jax.ShapeDtypeStruct((_N_PAD,), jnp.uint32)] * 2 + \
           [jax.ShapeDtypeStruct((_N_PAD,), jnp.int32)] * 6
    return pl.pallas_call(
        _hash_body,
        grid=grid,
        in_specs=[
            pl.BlockSpec((_ROW_BLK, _D), lambda i: (i, 0)),
            pl.BlockSpec((1, _D), lambda i: (0, 0)),
            pl.BlockSpec((1, _D), lambda i: (0, 0)),
        ],
        out_specs=[vec_spec] * 8,
        out_shape=outs,
    )(x_bits_pad, _R_LO, _R_HI)


def _scatter_body(src_hbm, dst_hbm, glimbs_hbm, zeros_hbm, out_hbm,
                  src_v, dst_v, buf0, buf1, acc, sem0, sem1):
    cid = lax.axis_index("c")
    sid = lax.axis_index("s")
    wid = cid * 16 + sid
    pltpu.sync_copy(src_hbm.at[wid], src_v)
    pltpu.sync_copy(dst_hbm.at[wid], dst_v)

    @pl.when(sid == 0)
    def _():
        pltpu.sync_copy(zeros_hbm, acc)

    plsc.subcore_barrier()

    pltpu.async_copy(glimbs_hbm.at[src_v.at[0]], buf0, sem0)

    def pair(t, carry):
        j = t * 2
        pltpu.async_copy(glimbs_hbm.at[src_v.at[j + 1]], buf1, sem1)
        pltpu.make_async_copy(glimbs_hbm.at[src_v.at[j]], buf0, sem0).wait()
        pltpu.sync_copy(buf0, acc.at[dst_v.at[j]], add=True)

        @pl.when(j + 2 < _NCHUNK)
        def _():
            pltpu.async_copy(glimbs_hbm.at[src_v.at[j + 2]], buf0, sem0)

        pltpu.make_async_copy(glimbs_hbm.at[src_v.at[j + 1]], buf1, sem1).wait()
        pltpu.sync_copy(buf1, acc.at[dst_v.at[j + 1]], add=True)
        return carry

    lax.fori_loop(jnp.int32(0), jnp.int32(_NCHUNK // 2), pair, jnp.int32(0))
    plsc.subcore_barrier()

    @pl.when(sid == 0)
    def _():
        pltpu.sync_copy(acc, out_hbm.at[cid])


@functools.cache
def _make_scatter():
    return functools.partial(
        pl.kernel,
        out_type=jax.ShapeDtypeStruct((2, _N_PAD, _GW), jnp.int32),
        mesh=plsc.VectorSubcoreMesh(core_axis_name="c", subcore_axis_name="s"),
        scratch_types=[
            pltpu.VMEM((_NCHUNK, _CHUNK), jnp.int32),
            pltpu.VMEM((_NCHUNK, _CHUNK), jnp.int32),
            pltpu.VMEM((_CHUNK, _GW), jnp.int32),
            pltpu.VMEM((_CHUNK, _GW), jnp.int32),
            pltpu.VMEM_SHARED((_N_PAD, _GW), jnp.int32),
            pltpu.SemaphoreType.DMA,
            pltpu.SemaphoreType.DMA,
        ],
        compiler_params=pltpu.CompilerParams(use_tc_tiling_on_sc=False),
    )(_scatter_body)


def _run_scatter(src, dst, glimbs, zeros):
    return _make_scatter()(src, dst, glimbs, zeros)


def _combine_body(hlo_ref, hhi_ref, pa_ref, pb_ref, valid_ref,
                  siglo_ref, sighi_ref):
    s = pa_ref[...] + pb_ref[...]
    s0 = s[0, :]
    s1 = s[1, :]
    s2 = s[2, :]
    s3 = s[3, :]
    s4 = s[4, :]
    s5 = s[5, :]
    lo, hi = s0, jnp.zeros_like(s0)
    lo, hi = _add64(lo, hi, s1 << 12, s1 >> 20)
    lo, hi = _add64(lo, hi, s2 << 24, s2 >> 8)
    hi = hi + (s3 << 4) + (s4 << 16) + (s5 << 28)
    hlo = hlo_ref[...]
    hhi = hhi_ref[...]
    tlo, thi = _mul64(hlo, hhi, _CK_LO, _CK_HI)
    tlo, thi = _add64(tlo, thi, lo, hi)
    siglo, sighi = _mix64(tlo, thi)
    vb = valid_ref[...] != 0
    siglo_ref[...] = jnp.where(vb, siglo, np.uint32(0xFFFFFFFF))
    sighi_ref[...] = jnp.where(vb, sighi, np.uint32(0xFFFFFFFF))


def _run_combine(hlo, hhi, p0, p1, valid):
    grid = (_N_PAD // _ROW_BLK,)
    vec_spec = pl.BlockSpec((_ROW_BLK,), lambda i: (i,))
    tab_spec = pl.BlockSpec((_GW, _ROW_BLK), lambda i: (0, i))
    return pl.pallas_call(
        _combine_body,
        grid=grid,
        in_specs=[vec_spec, vec_spec, tab_spec, tab_spec, vec_spec],
        out_specs=[vec_spec, vec_spec],
        out_shape=[jax.ShapeDtypeStruct((_N_PAD,), jnp.uint32)] * 2,
    )(hlo, hhi, p0, p1, valid)


_FB_ROWS = 40
_FB = _FB_ROWS * _LANE
_FNB = _N_PAD // _FB
_UMAX = np.uint32(0xFFFFFFFF)


def _first_body(jlo_ref, jhi_ref, slo_ref, shi_ref, mlo_ref, mhi_ref):
    jb = pl.program_id(0)
    jl = jlo_ref[...]
    jh = jhi_ref[...]
    jg = jb * _FB + lax.broadcasted_iota(jnp.int32, (_FB_ROWS, _LANE), 0) * _LANE \
        + lax.broadcasted_iota(jnp.int32, (_FB_ROWS, _LANE), 1)

    def body(k, acc):
        kl = slo_ref[0, k]
        kh = shi_ref[0, k]
        m = (jh == kh) & (jl == kl) & (k < jg)
        return acc + m.astype(jnp.int32)

    upper = (jb + 1) * _FB
    acc = lax.fori_loop(jnp.int32(0), upper, body,
                        jnp.zeros((_FB_ROWS, _LANE), jnp.int32))
    first = acc == 0
    mlo_ref[...] = jnp.where(first, jl, _UMAX)
    mhi_ref[...] = jnp.where(first, jh, _UMAX)


def _run_first(sig2_lo, sig2_hi, sflat_lo, sflat_hi):
    blk = pl.BlockSpec((_FB_ROWS, _LANE), lambda b: (b, 0))
    smem = pl.BlockSpec(memory_space=pltpu.SMEM)
    return pl.pallas_call(
        _first_body,
        grid=(_FNB,),
        in_specs=[blk, blk, smem, smem],
        out_specs=[blk, blk],
        out_shape=[jax.ShapeDtypeStruct((_JB, _LANE), jnp.uint32)] * 2,
    )(sig2_lo, sig2_hi, sflat_lo, sflat_hi)


def _rank_body(ilo_ref, ihi_ref, slo_ref, shi_ref, inv_ref):
    il = ilo_ref[...]
    ih = ihi_ref[...]

    def body(j, acc):
        jl = slo_ref[0, j]
        jh = shi_ref[0, j]
        m = (jh < ih) | ((jh == ih) & (jl < il))
        return acc + m.astype(jnp.int32)

    acc = lax.fori_loop(0, _N_PAD, body,
                        jnp.zeros((_JB, _LANE), jnp.int32), unroll=4)
    inv_ref[...] = acc


def _run_rank(sig2_lo, sig2_hi, mflat_lo, mflat_hi):
    blk = pl.BlockSpec((_JB, _LANE), lambda: (0, 0))
    smem = pl.BlockSpec(memory_space=pltpu.SMEM)
    return pl.pallas_call(
        _rank_body,
        in_specs=[blk, blk, smem, smem],
        out_specs=blk,
        out_shape=jax.ShapeDtypeStruct((_JB, _LANE), jnp.int32),
    )(sig2_lo, sig2_hi, mflat_lo, mflat_hi)


def kernel(x, edge_index):
    with _jax_config.enable_x64(False):
        inv32 = _kernel_impl(x, edge_index)
    return inv32.astype(jnp.int64)


def _kernel_impl(x, edge_index):
    x_bits = lax.bitcast_convert_type(x, jnp.uint32)
    x_bits = jnp.concatenate(
        [x_bits, jnp.zeros((_N_PAD - _N, _D), jnp.uint32)], axis=0)

    hlo, hhi, l0, l1, l2, l3, l4, l5 = _run_hash(x_bits)

    pad_cols = jnp.zeros((_N_PAD, _GW - 6), jnp.int32)
    glimbs = jnp.concatenate(
        [jnp.stack([l0, l1, l2, l3, l4, l5], axis=1), pad_cols], axis=1)

    ei = edge_index.astype(jnp.int32)
    pad_e = jnp.full((_E_PAD - _E,), _TRASH, jnp.int32)
    src = jnp.concatenate([ei[0], pad_e]).reshape(_NUM_TILES, _NCHUNK, _CHUNK)
    dst = jnp.concatenate([ei[1], pad_e]).reshape(_NUM_TILES, _NCHUNK, _CHUNK)

    zeros = jnp.zeros((_N_PAD, _GW), jnp.int32)
    partials = _run_scatter(src, dst, glimbs, zeros)
    p0 = lax.bitcast_convert_type(partials[0].T, jnp.uint32)
    p1 = lax.bitcast_convert_type(partials[1].T, jnp.uint32)

    valid = (jnp.arange(_N_PAD) < _N).astype(jnp.uint32)
    siglo, sighi = _run_combine(hlo, hhi, p0, p1, valid)

    sig2_lo = siglo.reshape(_JB, _LANE)
    sig2_hi = sighi.reshape(_JB, _LANE)
    sflat_lo = siglo.reshape(1, _N_PAD)
    sflat_hi = sighi.reshape(1, _N_PAD)

    mlo, mhi = _run_first(sig2_lo, sig2_hi, sflat_lo, sflat_hi)
    inv = _run_rank(sig2_lo, sig2_hi,
                    mlo.reshape(1, _N_PAD), mhi.reshape(1, _N_PAD))

    return inv.reshape(-1)[:_N]


# speedup vs baseline: 60.7745x; 1.0247x over previous
import functools
import numpy as np
import jax
import jax.numpy as jnp
from jax import lax
from jax._src import config as _jax_config
from jax.experimental import pallas as pl
from jax.experimental.pallas import tpu as pltpu
from jax.experimental.pallas import tpu_sc as plsc

_N = 10000
_D = 128
_E = 320000

_LANE = 128
_N_PAD = 10240
_ROW_BLK = 256
_JB = _N_PAD // _LANE

_NUM_TILES = 32
_CHUNK = 128
_E_TILE = 10240
_NCHUNK = _E_TILE // _CHUNK
_E_PAD = _NUM_TILES * _E_TILE
_TRASH = _N
_GW = 8

_rng_k = np.random.default_rng(42)
_R64 = _rng_k.integers(0, 2**64, size=(_D,), dtype=np.uint64) | np.uint64(1)
_R_LO = (_R64 & np.uint64(0xFFFFFFFF)).astype(np.uint32).reshape(1, _D)
_R_HI = (_R64 >> np.uint64(32)).astype(np.uint32).reshape(1, _D)

_M16 = np.uint32(0xFFFF)
_M12 = np.uint32(0xFFF)

_C1_LO, _C1_HI = np.uint32(0x7F4A7C15), np.uint32(0x9E3779B9)
_C2_LO, _C2_HI = np.uint32(0x1CE4E5B9), np.uint32(0xBF58476D)
_C3_LO, _C3_HI = np.uint32(0x133111EB), np.uint32(0x94D049BB)
_CX_LO, _CX_HI = np.uint32(0x6659FD93), np.uint32(0xD6E8FEB8)
_CK_LO, _CK_HI = np.uint32(0x000001B3), np.uint32(0x00000100)


def _mulhi32(a, b):
    al = a & _M16
    ah = a >> 16
    bl = b & _M16
    bh = b >> 16
    ll = al * bl
    lh = al * bh
    hl = ah * bl
    hh = ah * bh
    mid = lh + (ll >> 16)
    mid2 = hl + (mid & _M16)
    return hh + (mid >> 16) + (mid2 >> 16)


def _mul64(alo, ahi, blo, bhi):
    lo = alo * blo
    hi = _mulhi32(alo, blo) + alo * bhi + ahi * blo
    return lo, hi


def _add64(alo, ahi, blo, bhi):
    lo = alo + blo
    carry = (lo < blo).astype(jnp.uint32)
    return lo, ahi + bhi + carry


def _shr64(lo, hi, k):
    return (lo >> k) | (hi << (32 - k)), hi >> k


def _mix64(lo, hi):
    lo, hi = _add64(lo, hi, _C1_LO, _C1_HI)
    slo, shi = _shr64(lo, hi, 30)
    lo, hi = lo ^ slo, hi ^ shi
    lo, hi = _mul64(lo, hi, _C2_LO, _C2_HI)
    slo, shi = _shr64(lo, hi, 27)
    lo, hi = lo ^ slo, hi ^ shi
    lo, hi = _mul64(lo, hi, _C3_LO, _C3_HI)
    slo, shi = _shr64(lo, hi, 31)
    return lo ^ slo, hi ^ shi


def _hash_body(xb_ref, rlo_ref, rhi_ref, hlo_ref, hhi_ref,
               l0, l1, l2, l3, l4, l5):
    b = xb_ref[...]
    rlo = rlo_ref[...]
    rhi = rhi_ref[...]
    tlo = b * rlo
    thi = _mulhi32(b, rlo) + b * rhi
    def _usum(v):
        s = jnp.sum(lax.bitcast_convert_type(v, jnp.int32), axis=1,
                    dtype=jnp.int32)
        return lax.bitcast_convert_type(s, jnp.uint32)

    sl = _usum(tlo & _M16)
    sh = _usum(tlo >> 16)
    acc_lo = sl + (sh << 16)
    carry = (sh + (sl >> 16)) >> 16
    acc_hi = _usum(thi) + carry
    hlo, hhi = _mix64(acc_lo, acc_hi)
    hlo_ref[...] = hlo
    hhi_ref[...] = hhi
    glo, ghi = _mix64(hlo ^ _CX_LO, hhi ^ _CX_HI)
    l0[...] = (glo & _M12).astype(jnp.int32)
    l1[...] = ((glo >> 12) & _M12).astype(jnp.int32)
    l2[...] = (((glo >> 24) | (ghi << 8)) & _M12).astype(jnp.int32)
    l3[...] = ((ghi >> 4) & _M12).astype(jnp.int32)
    l4[...] = ((ghi >> 16) & _M12).astype(jnp.int32)
    l5[...] = (ghi >> 28).astype(jnp.int32)


def _run_hash(x_bits_pad):
    grid = (_N_PAD // _ROW_BLK,)
    vec_spec = pl.BlockSpec((_ROW_BLK,), lambda i: (i,))
    outs = [jax.ShapeDtypeStruct((_N_PAD,), jnp.uint32)] * 2 + \
           [jax.ShapeDtypeStruct((_N_PAD,), jnp.int32)] * 6
    return pl.pallas_call(
        _hash_body,
        grid=grid,
        in_specs=[
            pl.BlockSpec((_ROW_BLK, _D), lambda i: (i, 0)),
            pl.BlockSpec((1, _D), lambda i: (0, 0)),
            pl.BlockSpec((1, _D), lambda i: (0, 0)),
        ],
        out_specs=[vec_spec] * 8,
        out_shape=outs,
    )(x_bits_pad, _R_LO, _R_HI)


def _scatter_body(src_hbm, dst_hbm, glimbs_hbm, zeros_hbm, out_hbm,
                  src_v, dst_v, buf0, buf1, acc, sem0, sem1):
    cid = lax.axis_index("c")
    sid = lax.axis_index("s")
    wid = cid * 16 + sid
    pltpu.sync_copy(src_hbm.at[wid], src_v)
    pltpu.sync_copy(dst_hbm.at[wid], dst_v)

    @pl.when(sid == 0)
    def _():
        pltpu.sync_copy(zeros_hbm, acc)

    plsc.subcore_barrier()

    pltpu.async_copy(glimbs_hbm.at[src_v.at[0]], buf0, sem0)

    def pair(t, carry):
        j = t * 2
        pltpu.async_copy(glimbs_hbm.at[src_v.at[j + 1]], buf1, sem1)
        pltpu.make_async_copy(glimbs_hbm.at[src_v.at[j]], buf0, sem0).wait()
        pltpu.sync_copy(buf0, acc.at[dst_v.at[j]], add=True)

        @pl.when(j + 2 < _NCHUNK)
        def _():
            pltpu.async_copy(glimbs_hbm.at[src_v.at[j + 2]], buf0, sem0)

        pltpu.make_async_copy(glimbs_hbm.at[src_v.at[j + 1]], buf1, sem1).wait()
        pltpu.sync_copy(buf1, acc.at[dst_v.at[j + 1]], add=True)
        return carry

    lax.fori_loop(jnp.int32(0), jnp.int32(_NCHUNK // 2), pair, jnp.int32(0))
    plsc.subcore_barrier()

    @pl.when(sid == 0)
    def _():
        pltpu.sync_copy(acc, out_hbm.at[cid])


@functools.cache
def _make_scatter():
    return functools.partial(
        pl.kernel,
        out_type=jax.ShapeDtypeStruct((2, _N_PAD, _GW), jnp.int32),
        mesh=plsc.VectorSubcoreMesh(core_axis_name="c", subcore_axis_name="s"),
        scratch_types=[
            pltpu.VMEM((_NCHUNK, _CHUNK), jnp.int32),
            pltpu.VMEM((_NCHUNK, _CHUNK), jnp.int32),
            pltpu.VMEM((_CHUNK, _GW), jnp.int32),
            pltpu.VMEM((_CHUNK, _GW), jnp.int32),
            pltpu.VMEM_SHARED((_N_PAD, _GW), jnp.int32),
            pltpu.SemaphoreType.DMA,
            pltpu.SemaphoreType.DMA,
        ],
        compiler_params=pltpu.CompilerParams(use_tc_tiling_on_sc=False),
    )(_scatter_body)


def _run_scatter(src, dst, glimbs, zeros):
    return _make_scatter()(src, dst, glimbs, zeros)


def _combine_body(hlo_ref, hhi_ref, pa_ref, pb_ref, valid_ref,
                  siglo_ref, sighi_ref):
    s = pa_ref[...] + pb_ref[...]
    s0 = s[0, :]
    s1 = s[1, :]
    s2 = s[2, :]
    s3 = s[3, :]
    s4 = s[4, :]
    s5 = s[5, :]
    lo, hi = s0, jnp.zeros_like(s0)
    lo, hi = _add64(lo, hi, s1 << 12, s1 >> 20)
    lo, hi = _add64(lo, hi, s2 << 24, s2 >> 8)
    hi = hi + (s3 << 4) + (s4 << 16) + (s5 << 28)
    hlo = hlo_ref[...]
    hhi = hhi_ref[...]
    tlo, thi = _mul64(hlo, hhi, _CK_LO, _CK_HI)
    tlo, thi = _add64(tlo, thi, lo, hi)
    siglo, sighi = _mix64(tlo, thi)
    vb = valid_ref[...] != 0
    siglo_ref[...] = jnp.where(vb, siglo, np.uint32(0xFFFFFFFF))
    sighi_ref[...] = jnp.where(vb, sighi, np.uint32(0xFFFFFFFF))


def _run_combine(hlo, hhi, p0, p1, valid):
    grid = (_N_PAD // _ROW_BLK,)
    vec_spec = pl.BlockSpec((_ROW_BLK,), lambda i: (i,))
    tab_spec = pl.BlockSpec((_GW, _ROW_BLK), lambda i: (0, i))
    return pl.pallas_call(
        _combine_body,
        grid=grid,
        in_specs=[vec_spec, vec_spec, tab_spec, tab_spec, vec_spec],
        out_specs=[vec_spec, vec_spec],
        out_shape=[jax.ShapeDtypeStruct((_N_PAD,), jnp.uint32)] * 2,
    )(hlo, hhi, p0, p1, valid)


_FB_ROWS = 40
_FB = _FB_ROWS * _LANE
_FNB = _N_PAD // _FB
_UMAX = np.uint32(0xFFFFFFFF)


def _first_body(jlo_ref, jhi_ref, slo_ref, shi_ref, mlo_ref, mhi_ref):
    jb = pl.program_id(0)
    jl = jlo_ref[...]
    jh = jhi_ref[...]
    jg = jb * _FB + lax.broadcasted_iota(jnp.int32, (_FB_ROWS, _LANE), 0) * _LANE \
        + lax.broadcasted_iota(jnp.int32, (_FB_ROWS, _LANE), 1)

    def body(k, acc):
        kl = slo_ref[0, k]
        kh = shi_ref[0, k]
        m = (jh == kh) & (jl == kl) & (k < jg)
        return acc + m.astype(jnp.int32)

    upper = (jb + 1) * _FB
    acc = lax.fori_loop(jnp.int32(0), upper, body,
                        jnp.zeros((_FB_ROWS, _LANE), jnp.int32))
    first = acc == 0
    mlo_ref[...] = jnp.where(first, jl, _UMAX)
    mhi_ref[...] = jnp.where(first, jh, _UMAX)


def _run_first(sig2_lo, sig2_hi, sflat_lo, sflat_hi):
    blk = pl.BlockSpec((_FB_ROWS, _LANE), lambda b: (b, 0))
    smem = pl.BlockSpec(memory_space=pltpu.SMEM)
    return pl.pallas_call(
        _first_body,
        grid=(_FNB,),
        in_specs=[blk, blk, smem, smem],
        out_specs=[blk, blk],
        out_shape=[jax.ShapeDtypeStruct((_JB, _LANE), jnp.uint32)] * 2,
    )(sig2_lo, sig2_hi, sflat_lo, sflat_hi)


def _rank_body(ilo_ref, ihi_ref, slo_ref, shi_ref, inv_ref):
    il = ilo_ref[...]
    ih = ihi_ref[...]

    def body(j, acc):
        jl = slo_ref[0, j]
        jh = shi_ref[0, j]
        m = (jh < ih) | ((jh == ih) & (jl < il))
        return acc + m.astype(jnp.int32)

    acc = lax.fori_loop(0, _N_PAD, body,
                        jnp.zeros((_JB, _LANE), jnp.int32), unroll=4)
    inv_ref[...] = acc


def _run_rank(sig2_lo, sig2_hi, mflat_lo, mflat_hi):
    blk = pl.BlockSpec((_JB, _LANE), lambda: (0, 0))
    smem = pl.BlockSpec(memory_space=pltpu.SMEM)
    return pl.pallas_call(
        _rank_body,
        in_specs=[blk, blk, smem, smem],
        out_specs=blk,
        out_shape=jax.ShapeDtypeStruct((_JB, _LANE), jnp.int32),
    )(sig2_lo, sig2_hi, mflat_lo, mflat_hi)


def kernel(x, edge_index):
    with _jax_config.enable_x64(False):
        inv32 = _kernel_impl(x, edge_index)
    return inv32.astype(jnp.int64)


def _kernel_impl(x, edge_index):
    x_bits = lax.bitcast_convert_type(x, jnp.uint32)
    x_bits = jnp.concatenate(
        [x_bits, jnp.zeros((_N_PAD - _N, _D), jnp.uint32)], axis=0)

    hlo, hhi, l0, l1, l2, l3, l4, l5 = _run_hash(x_bits)

    pad_cols = jnp.zeros((_N_PAD, _GW - 6), jnp.int32)
    glimbs = jnp.concatenate(
        [jnp.stack([l0, l1, l2, l3, l4, l5], axis=1), pad_cols], axis=1)

    ei = edge_index.astype(jnp.int32)
    pad_e = jnp.full((_E_PAD - _E,), _TRASH, jnp.int32)
    src = jnp.concatenate([ei[0], pad_e]).reshape(_NUM_TILES, _NCHUNK, _CHUNK)
    dst = jnp.concatenate([ei[1], pad_e]).reshape(_NUM_TILES, _NCHUNK, _CHUNK)

    zeros = jnp.zeros((_N_PAD, _GW), jnp.int32)
    partials = _run_scatter(src, dst, glimbs, zeros)
    p0 = lax.bitcast_convert_type(partials[0].T, jnp.uint32)
    p1 = lax.bitcast_convert_type(partials[1].T, jnp.uint32)

    valid = (jnp.arange(_N_PAD) < _N).astype(jnp.uint32)
    siglo, sighi = _run_combine(hlo, hhi, p0, p1, valid)

    sig2_lo = siglo.reshape(_JB, _LANE)
    sig2_hi = sighi.reshape(_JB, _LANE)
    sflat_lo = siglo.reshape(1, _N_PAD)
    sflat_hi = sighi.reshape(1, _N_PAD)

    mlo, mhi = _run_first(sig2_lo, sig2_hi, sflat_lo, sflat_hi)
    inv = _run_rank(sig2_lo, sig2_hi,
                    mlo.reshape(1, _N_PAD), mhi.reshape(1, _N_PAD))

    return inv.reshape(-1)[:_N]


# speedup vs baseline: 68.4903x; 1.1270x over previous
import functools
import numpy as np
import jax
import jax.numpy as jnp
from jax import lax
from jax._src import config as _jax_config
from jax.experimental import pallas as pl
from jax.experimental.pallas import tpu as pltpu
from jax.experimental.pallas import tpu_sc as plsc

_N = 10000
_D = 128
_E = 320000

_LANE = 128
_N_PAD = 10240
_ROW_BLK = 256
_JB = _N_PAD // _LANE

_NUM_TILES = 32
_CHUNK = 128
_E_TILE = 10240
_NCHUNK = _E_TILE // _CHUNK
_E_PAD = _NUM_TILES * _E_TILE
_TRASH = _N
_GW = 8

_rng_k = np.random.default_rng(42)
_R64 = _rng_k.integers(0, 2**64, size=(_D,), dtype=np.uint64) | np.uint64(1)
_R_LO = (_R64 & np.uint64(0xFFFFFFFF)).astype(np.uint32).reshape(1, _D)
_R_HI = (_R64 >> np.uint64(32)).astype(np.uint32).reshape(1, _D)

_M16 = np.uint32(0xFFFF)
_M12 = np.uint32(0xFFF)

_C1_LO, _C1_HI = np.uint32(0x7F4A7C15), np.uint32(0x9E3779B9)
_C2_LO, _C2_HI = np.uint32(0x1CE4E5B9), np.uint32(0xBF58476D)
_C3_LO, _C3_HI = np.uint32(0x133111EB), np.uint32(0x94D049BB)
_CX_LO, _CX_HI = np.uint32(0x6659FD93), np.uint32(0xD6E8FEB8)
_CK_LO, _CK_HI = np.uint32(0x000001B3), np.uint32(0x00000100)


def _mulhi32(a, b):
    al = a & _M16
    ah = a >> 16
    bl = b & _M16
    bh = b >> 16
    ll = al * bl
    lh = al * bh
    hl = ah * bl
    hh = ah * bh
    mid = lh + (ll >> 16)
    mid2 = hl + (mid & _M16)
    return hh + (mid >> 16) + (mid2 >> 16)


def _mul64(alo, ahi, blo, bhi):
    lo = alo * blo
    hi = _mulhi32(alo, blo) + alo * bhi + ahi * blo
    return lo, hi


def _add64(alo, ahi, blo, bhi):
    lo = alo + blo
    carry = (lo < blo).astype(jnp.uint32)
    return lo, ahi + bhi + carry


def _shr64(lo, hi, k):
    return (lo >> k) | (hi << (32 - k)), hi >> k


def _mix64(lo, hi):
    lo, hi = _add64(lo, hi, _C1_LO, _C1_HI)
    slo, shi = _shr64(lo, hi, 30)
    lo, hi = lo ^ slo, hi ^ shi
    lo, hi = _mul64(lo, hi, _C2_LO, _C2_HI)
    slo, shi = _shr64(lo, hi, 27)
    lo, hi = lo ^ slo, hi ^ shi
    lo, hi = _mul64(lo, hi, _C3_LO, _C3_HI)
    slo, shi = _shr64(lo, hi, 31)
    return lo ^ slo, hi ^ shi


def _hash_body(xb_ref, rlo_ref, rhi_ref, hlo_ref, hhi_ref,
               l0, l1, l2, l3, l4, l5):
    b = xb_ref[...]
    rlo = rlo_ref[...]
    rhi = rhi_ref[...]
    tlo = b * rlo
    thi = _mulhi32(b, rlo) + b * rhi
    def _usum(v):
        s = jnp.sum(lax.bitcast_convert_type(v, jnp.int32), axis=1,
                    dtype=jnp.int32)
        return lax.bitcast_convert_type(s, jnp.uint32)

    sl = _usum(tlo & _M16)
    sh = _usum(tlo >> 16)
    acc_lo = sl + (sh << 16)
    carry = (sh + (sl >> 16)) >> 16
    acc_hi = _usum(thi) + carry
    hlo, hhi = _mix64(acc_lo, acc_hi)
    hlo_ref[...] = hlo
    hhi_ref[...] = hhi
    glo, ghi = _mix64(hlo ^ _CX_LO, hhi ^ _CX_HI)
    l0[...] = (glo & _M12).astype(jnp.int32)
    l1[...] = ((glo >> 12) & _M12).astype(jnp.int32)
    l2[...] = (((glo >> 24) | (ghi << 8)) & _M12).astype(jnp.int32)
    l3[...] = ((ghi >> 4) & _M12).astype(jnp.int32)
    l4[...] = ((ghi >> 16) & _M12).astype(jnp.int32)
    l5[...] = (ghi >> 28).astype(jnp.int32)


def _run_hash(x_bits_pad):
    grid = (_N_PAD // _ROW_BLK,)
    vec_spec = pl.BlockSpec((_ROW_BLK,), lambda i: (i,))
    outs = [jax.ShapeDtypeStruct((_N_PAD,), jnp.uint32)] * 2 + \
           [jax.ShapeDtypeStruct((_N_PAD,), jnp.int32)] * 6
    return pl.pallas_call(
        _hash_body,
        grid=grid,
        in_specs=[
            pl.BlockSpec((_ROW_BLK, _D), lambda i: (i, 0)),
            pl.BlockSpec((1, _D), lambda i: (0, 0)),
            pl.BlockSpec((1, _D), lambda i: (0, 0)),
        ],
        out_specs=[vec_spec] * 8,
        out_shape=outs,
    )(x_bits_pad, _R_LO, _R_HI)


def _scatter_body(src_hbm, dst_hbm, glimbs_hbm, zeros_hbm, out_hbm,
                  src_v, dst_v, buf0, buf1, acc, sem0, sem1):
    cid = lax.axis_index("c")
    sid = lax.axis_index("s")
    wid = cid * 16 + sid
    pltpu.sync_copy(src_hbm.at[wid], src_v)
    pltpu.sync_copy(dst_hbm.at[wid], dst_v)

    @pl.when(sid == 0)
    def _():
        pltpu.sync_copy(zeros_hbm, acc)

    plsc.subcore_barrier()

    pltpu.async_copy(glimbs_hbm.at[src_v.at[0]], buf0, sem0)

    def pair(t, carry):
        j = t * 2
        pltpu.async_copy(glimbs_hbm.at[src_v.at[j + 1]], buf1, sem1)
        pltpu.make_async_copy(glimbs_hbm.at[src_v.at[j]], buf0, sem0).wait()
        pltpu.sync_copy(buf0, acc.at[dst_v.at[j]], add=True)

        @pl.when(j + 2 < _NCHUNK)
        def _():
            pltpu.async_copy(glimbs_hbm.at[src_v.at[j + 2]], buf0, sem0)

        pltpu.make_async_copy(glimbs_hbm.at[src_v.at[j + 1]], buf1, sem1).wait()
        pltpu.sync_copy(buf1, acc.at[dst_v.at[j + 1]], add=True)
        return carry

    lax.fori_loop(jnp.int32(0), jnp.int32(_NCHUNK // 2), pair, jnp.int32(0))
    plsc.subcore_barrier()

    @pl.when(sid == 0)
    def _():
        pltpu.sync_copy(acc, out_hbm.at[cid])


@functools.cache
def _make_scatter():
    return functools.partial(
        pl.kernel,
        out_type=jax.ShapeDtypeStruct((2, _N_PAD, _GW), jnp.int32),
        mesh=plsc.VectorSubcoreMesh(core_axis_name="c", subcore_axis_name="s"),
        scratch_types=[
            pltpu.VMEM((_NCHUNK, _CHUNK), jnp.int32),
            pltpu.VMEM((_NCHUNK, _CHUNK), jnp.int32),
            pltpu.VMEM((_CHUNK, _GW), jnp.int32),
            pltpu.VMEM((_CHUNK, _GW), jnp.int32),
            pltpu.VMEM_SHARED((_N_PAD, _GW), jnp.int32),
            pltpu.SemaphoreType.DMA,
            pltpu.SemaphoreType.DMA,
        ],
        compiler_params=pltpu.CompilerParams(use_tc_tiling_on_sc=False),
    )(_scatter_body)


def _run_scatter(src, dst, glimbs, zeros):
    return _make_scatter()(src, dst, glimbs, zeros)


def _combine_body(hlo_ref, hhi_ref, pa_ref, pb_ref, valid_ref,
                  siglo_ref, sighi_ref):
    s = pa_ref[...] + pb_ref[...]
    s0 = s[0, :]
    s1 = s[1, :]
    s2 = s[2, :]
    s3 = s[3, :]
    s4 = s[4, :]
    s5 = s[5, :]
    lo, hi = s0, jnp.zeros_like(s0)
    lo, hi = _add64(lo, hi, s1 << 12, s1 >> 20)
    lo, hi = _add64(lo, hi, s2 << 24, s2 >> 8)
    hi = hi + (s3 << 4) + (s4 << 16) + (s5 << 28)
    hlo = hlo_ref[...]
    hhi = hhi_ref[...]
    tlo, thi = _mul64(hlo, hhi, _CK_LO, _CK_HI)
    tlo, thi = _add64(tlo, thi, lo, hi)
    siglo, sighi = _mix64(tlo, thi)
    vb = valid_ref[...] != 0
    siglo_ref[...] = jnp.where(vb, siglo, np.uint32(0xFFFFFFFF))
    sighi_ref[...] = jnp.where(vb, sighi, np.uint32(0xFFFFFFFF))


def _run_combine(hlo, hhi, p0, p1, valid):
    grid = (_N_PAD // _ROW_BLK,)
    vec_spec = pl.BlockSpec((_ROW_BLK,), lambda i: (i,))
    tab_spec = pl.BlockSpec((_GW, _ROW_BLK), lambda i: (0, i))
    return pl.pallas_call(
        _combine_body,
        grid=grid,
        in_specs=[vec_spec, vec_spec, tab_spec, tab_spec, vec_spec],
        out_specs=[vec_spec, vec_spec],
        out_shape=[jax.ShapeDtypeStruct((_N_PAD,), jnp.uint32)] * 2,
    )(hlo, hhi, p0, p1, valid)


_FB_ROWS = 40
_FB = _FB_ROWS * _LANE
_FNB = _N_PAD // _FB
_UMAX = np.uint32(0xFFFFFFFF)


def _first_body(jlo_ref, jhi_ref, slo_ref, shi_ref, mlo_ref, mhi_ref):
    jb = pl.program_id(0)
    jl = jlo_ref[...]
    jh = jhi_ref[...]
    jg = jb * _FB + lax.broadcasted_iota(jnp.int32, (_FB_ROWS, _LANE), 0) * _LANE \
        + lax.broadcasted_iota(jnp.int32, (_FB_ROWS, _LANE), 1)

    def body(k, acc):
        kl = slo_ref[0, k]
        kh = shi_ref[0, k]
        m = (jh == kh) & (jl == kl) & (k < jg)
        return acc + m.astype(jnp.int32)

    zero = jnp.zeros((_FB_ROWS, _LANE), jnp.int32)
    acc = lax.switch(jb, [
        lambda u=u: lax.fori_loop(0, (u + 1) * _FB, body, zero, unroll=4)
        for u in range(_FNB)
    ])
    first = acc == 0
    mlo_ref[...] = jnp.where(first, jl, _UMAX)
    mhi_ref[...] = jnp.where(first, jh, _UMAX)


def _run_first(sig2_lo, sig2_hi, sflat_lo, sflat_hi):
    blk = pl.BlockSpec((_FB_ROWS, _LANE), lambda b: (b, 0))
    smem = pl.BlockSpec(memory_space=pltpu.SMEM)
    return pl.pallas_call(
        _first_body,
        grid=(_FNB,),
        in_specs=[blk, blk, smem, smem],
        out_specs=[blk, blk],
        out_shape=[jax.ShapeDtypeStruct((_JB, _LANE), jnp.uint32)] * 2,
    )(sig2_lo, sig2_hi, sflat_lo, sflat_hi)


def _rank_body(ilo_ref, ihi_ref, slo_ref, shi_ref, inv_ref):
    il = ilo_ref[...]
    ih = ihi_ref[...]

    def body(j, acc):
        jl = slo_ref[0, j]
        jh = shi_ref[0, j]
        m = (jh < ih) | ((jh == ih) & (jl < il))
        return acc + m.astype(jnp.int32)

    acc = lax.fori_loop(0, _N_PAD, body,
                        jnp.zeros((_JB, _LANE), jnp.int32), unroll=8)
    inv_ref[...] = acc


def _run_rank(sig2_lo, sig2_hi, mflat_lo, mflat_hi):
    blk = pl.BlockSpec((_JB, _LANE), lambda: (0, 0))
    smem = pl.BlockSpec(memory_space=pltpu.SMEM)
    return pl.pallas_call(
        _rank_body,
        in_specs=[blk, blk, smem, smem],
        out_specs=blk,
        out_shape=jax.ShapeDtypeStruct((_JB, _LANE), jnp.int32),
    )(sig2_lo, sig2_hi, mflat_lo, mflat_hi)


def kernel(x, edge_index):
    with _jax_config.enable_x64(False):
        inv32 = _kernel_impl(x, edge_index)
    return inv32.astype(jnp.int64)


def _kernel_impl(x, edge_index):
    x_bits = lax.bitcast_convert_type(x, jnp.uint32)
    x_bits = jnp.concatenate(
        [x_bits, jnp.zeros((_N_PAD - _N, _D), jnp.uint32)], axis=0)

    hlo, hhi, l0, l1, l2, l3, l4, l5 = _run_hash(x_bits)

    pad_cols = jnp.zeros((_N_PAD, _GW - 6), jnp.int32)
    glimbs = jnp.concatenate(
        [jnp.stack([l0, l1, l2, l3, l4, l5], axis=1), pad_cols], axis=1)

    ei = edge_index.astype(jnp.int32)
    pad_e = jnp.full((_E_PAD - _E,), _TRASH, jnp.int32)
    src = jnp.concatenate([ei[0], pad_e]).reshape(_NUM_TILES, _NCHUNK, _CHUNK)
    dst = jnp.concatenate([ei[1], pad_e]).reshape(_NUM_TILES, _NCHUNK, _CHUNK)

    zeros = jnp.zeros((_N_PAD, _GW), jnp.int32)
    partials = _run_scatter(src, dst, glimbs, zeros)
    p0 = lax.bitcast_convert_type(partials[0].T, jnp.uint32)
    p1 = lax.bitcast_convert_type(partials[1].T, jnp.uint32)

    valid = (jnp.arange(_N_PAD) < _N).astype(jnp.uint32)
    siglo, sighi = _run_combine(hlo, hhi, p0, p1, valid)

    sig2_lo = siglo.reshape(_JB, _LANE)
    sig2_hi = sighi.reshape(_JB, _LANE)
    sflat_lo = siglo.reshape(1, _N_PAD)
    sflat_hi = sighi.reshape(1, _N_PAD)

    mlo, mhi = _run_first(sig2_lo, sig2_hi, sflat_lo, sflat_hi)
    inv = _run_rank(sig2_lo, sig2_hi,
                    mlo.reshape(1, _N_PAD), mhi.reshape(1, _N_PAD))

    return inv.reshape(-1)[:_N]


# speedup vs baseline: 68.6577x; 1.0024x over previous
import functools
import numpy as np
import jax
import jax.numpy as jnp
from jax import lax
from jax._src import config as _jax_config
from jax.experimental import pallas as pl
from jax.experimental.pallas import tpu as pltpu
from jax.experimental.pallas import tpu_sc as plsc

_N = 10000
_D = 128
_E = 320000

_LANE = 128
_N_PAD = 10240
_ROW_BLK = 256
_JB = _N_PAD // _LANE

_NUM_TILES = 32
_CHUNK = 128
_E_TILE = 10240
_NCHUNK = _E_TILE // _CHUNK
_E_PAD = _NUM_TILES * _E_TILE
_TRASH = _N
_GW = 8

_rng_k = np.random.default_rng(42)
_R64 = _rng_k.integers(0, 2**64, size=(_D,), dtype=np.uint64) | np.uint64(1)
_R_LO = (_R64 & np.uint64(0xFFFFFFFF)).astype(np.uint32).reshape(1, _D)
_R_HI = (_R64 >> np.uint64(32)).astype(np.uint32).reshape(1, _D)

_M16 = np.uint32(0xFFFF)
_M12 = np.uint32(0xFFF)

_C1_LO, _C1_HI = np.uint32(0x7F4A7C15), np.uint32(0x9E3779B9)
_C2_LO, _C2_HI = np.uint32(0x1CE4E5B9), np.uint32(0xBF58476D)
_C3_LO, _C3_HI = np.uint32(0x133111EB), np.uint32(0x94D049BB)
_CX_LO, _CX_HI = np.uint32(0x6659FD93), np.uint32(0xD6E8FEB8)
_CK_LO, _CK_HI = np.uint32(0x000001B3), np.uint32(0x00000100)


def _mulhi32(a, b):
    al = a & _M16
    ah = a >> 16
    bl = b & _M16
    bh = b >> 16
    ll = al * bl
    lh = al * bh
    hl = ah * bl
    hh = ah * bh
    mid = lh + (ll >> 16)
    mid2 = hl + (mid & _M16)
    return hh + (mid >> 16) + (mid2 >> 16)


def _mul64(alo, ahi, blo, bhi):
    lo = alo * blo
    hi = _mulhi32(alo, blo) + alo * bhi + ahi * blo
    return lo, hi


def _add64(alo, ahi, blo, bhi):
    lo = alo + blo
    carry = (lo < blo).astype(jnp.uint32)
    return lo, ahi + bhi + carry


def _shr64(lo, hi, k):
    return (lo >> k) | (hi << (32 - k)), hi >> k


def _mix64(lo, hi):
    lo, hi = _add64(lo, hi, _C1_LO, _C1_HI)
    slo, shi = _shr64(lo, hi, 30)
    lo, hi = lo ^ slo, hi ^ shi
    lo, hi = _mul64(lo, hi, _C2_LO, _C2_HI)
    slo, shi = _shr64(lo, hi, 27)
    lo, hi = lo ^ slo, hi ^ shi
    lo, hi = _mul64(lo, hi, _C3_LO, _C3_HI)
    slo, shi = _shr64(lo, hi, 31)
    return lo ^ slo, hi ^ shi


def _hash_body(xb_ref, rlo_ref, rhi_ref, hlo_ref, hhi_ref,
               l0, l1, l2, l3, l4, l5):
    b = xb_ref[...]
    rlo = rlo_ref[...]
    rhi = rhi_ref[...]
    tlo = b * rlo
    thi = _mulhi32(b, rlo) + b * rhi
    def _usum(v):
        s = jnp.sum(lax.bitcast_convert_type(v, jnp.int32), axis=1,
                    dtype=jnp.int32)
        return lax.bitcast_convert_type(s, jnp.uint32)

    sl = _usum(tlo & _M16)
    sh = _usum(tlo >> 16)
    acc_lo = sl + (sh << 16)
    carry = (sh + (sl >> 16)) >> 16
    acc_hi = _usum(thi) + carry
    hlo, hhi = _mix64(acc_lo, acc_hi)
    hlo_ref[...] = hlo
    hhi_ref[...] = hhi
    glo, ghi = _mix64(hlo ^ _CX_LO, hhi ^ _CX_HI)
    l0[...] = (glo & _M12).astype(jnp.int32)
    l1[...] = ((glo >> 12) & _M12).astype(jnp.int32)
    l2[...] = (((glo >> 24) | (ghi << 8)) & _M12).astype(jnp.int32)
    l3[...] = ((ghi >> 4) & _M12).astype(jnp.int32)
    l4[...] = ((ghi >> 16) & _M12).astype(jnp.int32)
    l5[...] = (ghi >> 28).astype(jnp.int32)


def _run_hash(x_bits_pad):
    grid = (_N_PAD // _ROW_BLK,)
    vec_spec = pl.BlockSpec((_ROW_BLK,), lambda i: (i,))
    outs = [jax.ShapeDtypeStruct((_N_PAD,), jnp.uint32)] * 2 + \
           [jax.ShapeDtypeStruct((_N_PAD,), jnp.int32)] * 6
    return pl.pallas_call(
        _hash_body,
        grid=grid,
        in_specs=[
            pl.BlockSpec((_ROW_BLK, _D), lambda i: (i, 0)),
            pl.BlockSpec((1, _D), lambda i: (0, 0)),
            pl.BlockSpec((1, _D), lambda i: (0, 0)),
        ],
        out_specs=[vec_spec] * 8,
        out_shape=outs,
    )(x_bits_pad, _R_LO, _R_HI)


def _scatter_body(src_hbm, dst_hbm, glimbs_hbm, zeros_hbm, out_hbm,
                  src_v, dst_v, b0, b1, b2, b3, b4, b5, b6, b7,
                  acc, gsem, ssem):
    bufs = (b0, b1, b2, b3, b4, b5, b6, b7)
    cid = lax.axis_index("c")
    sid = lax.axis_index("s")
    wid = cid * 16 + sid
    pltpu.sync_copy(src_hbm.at[wid], src_v)
    pltpu.sync_copy(dst_hbm.at[wid], dst_v)

    @pl.when(sid == 0)
    def _():
        pltpu.sync_copy(zeros_hbm, acc)

    plsc.subcore_barrier()

    def g_start(j, b):
        pltpu.async_copy(glimbs_hbm.at[src_v.at[j]], bufs[b], gsem)

    def g_wait(j, b):
        pltpu.make_async_copy(glimbs_hbm.at[src_v.at[j]], bufs[b], gsem).wait()

    def s_start(j, b):
        pltpu.async_copy(bufs[b], acc.at[dst_v.at[j]], ssem, add=True)

    def s_wait(j, b):
        pltpu.make_async_copy(bufs[b], acc.at[dst_v.at[j]], ssem).wait()

    for k in range(4):
        g_start(jnp.int32(k), k)

    def ring(t, carry):
        j0 = t * 8
        for k in range(8):
            j = j0 + k
            g_wait(j, k)
            s_start(j, k)

            @pl.when(j >= 4)
            def _():
                s_wait(j - 4, (k + 4) % 8)

            @pl.when(j + 4 < _NCHUNK)
            def _():
                g_start(j + 4, (k + 4) % 8)
        return carry

    lax.fori_loop(jnp.int32(0), jnp.int32(_NCHUNK // 8), ring, jnp.int32(0))
    for k in range(4):
        jj = jnp.int32(_NCHUNK - 4 + k)
        s_wait(jj, (_NCHUNK - 4 + k) % 8)
    plsc.subcore_barrier()

    @pl.when(sid == 0)
    def _():
        pltpu.sync_copy(acc, out_hbm.at[cid])


@functools.cache
def _make_scatter():
    return functools.partial(
        pl.kernel,
        out_type=jax.ShapeDtypeStruct((2, _N_PAD, _GW), jnp.int32),
        mesh=plsc.VectorSubcoreMesh(core_axis_name="c", subcore_axis_name="s"),
        scratch_types=[
            pltpu.VMEM((_NCHUNK, _CHUNK), jnp.int32),
            pltpu.VMEM((_NCHUNK, _CHUNK), jnp.int32),
            *([pltpu.VMEM((_CHUNK, _GW), jnp.int32)] * 8),
            pltpu.VMEM_SHARED((_N_PAD, _GW), jnp.int32),
            pltpu.SemaphoreType.DMA,
            pltpu.SemaphoreType.DMA,
        ],
        compiler_params=pltpu.CompilerParams(use_tc_tiling_on_sc=False),
    )(_scatter_body)


def _run_scatter(src, dst, glimbs, zeros):
    return _make_scatter()(src, dst, glimbs, zeros)


def _combine_body(hlo_ref, hhi_ref, pa_ref, pb_ref, valid_ref,
                  siglo_ref, sighi_ref):
    s = pa_ref[...] + pb_ref[...]
    s0 = s[0, :]
    s1 = s[1, :]
    s2 = s[2, :]
    s3 = s[3, :]
    s4 = s[4, :]
    s5 = s[5, :]
    lo, hi = s0, jnp.zeros_like(s0)
    lo, hi = _add64(lo, hi, s1 << 12, s1 >> 20)
    lo, hi = _add64(lo, hi, s2 << 24, s2 >> 8)
    hi = hi + (s3 << 4) + (s4 << 16) + (s5 << 28)
    hlo = hlo_ref[...]
    hhi = hhi_ref[...]
    tlo, thi = _mul64(hlo, hhi, _CK_LO, _CK_HI)
    tlo, thi = _add64(tlo, thi, lo, hi)
    siglo, sighi = _mix64(tlo, thi)
    vb = valid_ref[...] != 0
    siglo_ref[...] = jnp.where(vb, siglo, np.uint32(0xFFFFFFFF))
    sighi_ref[...] = jnp.where(vb, sighi, np.uint32(0xFFFFFFFF))


def _run_combine(hlo, hhi, p0, p1, valid):
    grid = (_N_PAD // _ROW_BLK,)
    vec_spec = pl.BlockSpec((_ROW_BLK,), lambda i: (i,))
    tab_spec = pl.BlockSpec((_GW, _ROW_BLK), lambda i: (0, i))
    return pl.pallas_call(
        _combine_body,
        grid=grid,
        in_specs=[vec_spec, vec_spec, tab_spec, tab_spec, vec_spec],
        out_specs=[vec_spec, vec_spec],
        out_shape=[jax.ShapeDtypeStruct((_N_PAD,), jnp.uint32)] * 2,
    )(hlo, hhi, p0, p1, valid)


_FB_ROWS = 40
_FB = _FB_ROWS * _LANE
_FNB = _N_PAD // _FB
_UMAX = np.uint32(0xFFFFFFFF)


def _first_body(jlo_ref, jhi_ref, slo_ref, shi_ref, mlo_ref, mhi_ref):
    jb = pl.program_id(0)
    jl = jlo_ref[...]
    jh = jhi_ref[...]
    jg = jb * _FB + lax.broadcasted_iota(jnp.int32, (_FB_ROWS, _LANE), 0) * _LANE \
        + lax.broadcasted_iota(jnp.int32, (_FB_ROWS, _LANE), 1)

    def body(k, acc):
        kl = slo_ref[0, k]
        kh = shi_ref[0, k]
        m = (jh == kh) & (jl == kl) & (k < jg)
        return acc + m.astype(jnp.int32)

    zero = jnp.zeros((_FB_ROWS, _LANE), jnp.int32)
    acc = lax.switch(jb, [
        lambda u=u: lax.fori_loop(0, (u + 1) * _FB, body, zero, unroll=4)
        for u in range(_FNB)
    ])
    first = acc == 0
    mlo_ref[...] = jnp.where(first, jl, _UMAX)
    mhi_ref[...] = jnp.where(first, jh, _UMAX)


def _run_first(sig2_lo, sig2_hi, sflat_lo, sflat_hi):
    blk = pl.BlockSpec((_FB_ROWS, _LANE), lambda b: (b, 0))
    smem = pl.BlockSpec(memory_space=pltpu.SMEM)
    return pl.pallas_call(
        _first_body,
        grid=(_FNB,),
        in_specs=[blk, blk, smem, smem],
        out_specs=[blk, blk],
        out_shape=[jax.ShapeDtypeStruct((_JB, _LANE), jnp.uint32)] * 2,
    )(sig2_lo, sig2_hi, sflat_lo, sflat_hi)


def _rank_body(ilo_ref, ihi_ref, slo_ref, shi_ref, inv_ref):
    il = ilo_ref[...]
    ih = ihi_ref[...]

    def body(j, acc):
        jl = slo_ref[0, j]
        jh = shi_ref[0, j]
        m = (jh < ih) | ((jh == ih) & (jl < il))
        return acc + m.astype(jnp.int32)

    acc = lax.fori_loop(0, _N_PAD, body,
                        jnp.zeros((_JB, _LANE), jnp.int32), unroll=8)
    inv_ref[...] = acc


def _run_rank(sig2_lo, sig2_hi, mflat_lo, mflat_hi):
    blk = pl.BlockSpec((_JB, _LANE), lambda: (0, 0))
    smem = pl.BlockSpec(memory_space=pltpu.SMEM)
    return pl.pallas_call(
        _rank_body,
        in_specs=[blk, blk, smem, smem],
        out_specs=blk,
        out_shape=jax.ShapeDtypeStruct((_JB, _LANE), jnp.int32),
    )(sig2_lo, sig2_hi, mflat_lo, mflat_hi)


def kernel(x, edge_index):
    with _jax_config.enable_x64(False):
        inv32 = _kernel_impl(x, edge_index)
    return inv32.astype(jnp.int64)


def _kernel_impl(x, edge_index):
    x_bits = lax.bitcast_convert_type(x, jnp.uint32)
    x_bits = jnp.concatenate(
        [x_bits, jnp.zeros((_N_PAD - _N, _D), jnp.uint32)], axis=0)

    hlo, hhi, l0, l1, l2, l3, l4, l5 = _run_hash(x_bits)

    pad_cols = jnp.zeros((_N_PAD, _GW - 6), jnp.int32)
    glimbs = jnp.concatenate(
        [jnp.stack([l0, l1, l2, l3, l4, l5], axis=1), pad_cols], axis=1)

    ei = edge_index.astype(jnp.int32)
    pad_e = jnp.full((_E_PAD - _E,), _TRASH, jnp.int32)
    src = jnp.concatenate([ei[0], pad_e]).reshape(_NUM_TILES, _NCHUNK, _CHUNK)
    dst = jnp.concatenate([ei[1], pad_e]).reshape(_NUM_TILES, _NCHUNK, _CHUNK)

    zeros = jnp.zeros((_N_PAD, _GW), jnp.int32)
    partials = _run_scatter(src, dst, glimbs, zeros)
    p0 = lax.bitcast_convert_type(partials[0].T, jnp.uint32)
    p1 = lax.bitcast_convert_type(partials[1].T, jnp.uint32)

    valid = (jnp.arange(_N_PAD) < _N).astype(jnp.uint32)
    siglo, sighi = _run_combine(hlo, hhi, p0, p1, valid)

    sig2_lo = siglo.reshape(_JB, _LANE)
    sig2_hi = sighi.reshape(_JB, _LANE)
    sflat_lo = siglo.reshape(1, _N_PAD)
    sflat_hi = sighi.reshape(1, _N_PAD)

    mlo, mhi = _run_first(sig2_lo, sig2_hi, sflat_lo, sflat_hi)
    inv = _run_rank(sig2_lo, sig2_hi,
                    mlo.reshape(1, _N_PAD), mhi.reshape(1, _N_PAD))

    return inv.reshape(-1)[:_N]


# speedup vs baseline: 70.2011x; 1.0225x over previous
import functools
import numpy as np
import jax
import jax.numpy as jnp
from jax import lax
from jax._src import config as _jax_config
from jax.experimental import pallas as pl
from jax.experimental.pallas import tpu as pltpu
from jax.experimental.pallas import tpu_sc as plsc

_N = 10000
_D = 128
_E = 320000

_LANE = 128
_N_PAD = 10240
_ROW_BLK = 256
_JB = _N_PAD // _LANE

_NUM_TILES = 32
_CHUNK = 128
_E_TILE = 10240
_NCHUNK = _E_TILE // _CHUNK
_E_PAD = _NUM_TILES * _E_TILE
_TRASH = _N
_GW = 8

_rng_k = np.random.default_rng(42)
_R64 = _rng_k.integers(0, 2**64, size=(_D,), dtype=np.uint64) | np.uint64(1)
_R_LO = (_R64 & np.uint64(0xFFFFFFFF)).astype(np.uint32).reshape(1, _D)
_R_HI = (_R64 >> np.uint64(32)).astype(np.uint32).reshape(1, _D)

_M16 = np.uint32(0xFFFF)
_M12 = np.uint32(0xFFF)

_C1_LO, _C1_HI = np.uint32(0x7F4A7C15), np.uint32(0x9E3779B9)
_C2_LO, _C2_HI = np.uint32(0x1CE4E5B9), np.uint32(0xBF58476D)
_C3_LO, _C3_HI = np.uint32(0x133111EB), np.uint32(0x94D049BB)
_CX_LO, _CX_HI = np.uint32(0x6659FD93), np.uint32(0xD6E8FEB8)
_CK_LO, _CK_HI = np.uint32(0x000001B3), np.uint32(0x00000100)


def _mulhi32(a, b):
    al = a & _M16
    ah = a >> 16
    bl = b & _M16
    bh = b >> 16
    ll = al * bl
    lh = al * bh
    hl = ah * bl
    hh = ah * bh
    mid = lh + (ll >> 16)
    mid2 = hl + (mid & _M16)
    return hh + (mid >> 16) + (mid2 >> 16)


def _mul64(alo, ahi, blo, bhi):
    lo = alo * blo
    hi = _mulhi32(alo, blo) + alo * bhi + ahi * blo
    return lo, hi


def _add64(alo, ahi, blo, bhi):
    lo = alo + blo
    carry = (lo < blo).astype(jnp.uint32)
    return lo, ahi + bhi + carry


def _shr64(lo, hi, k):
    return (lo >> k) | (hi << (32 - k)), hi >> k


def _mix64(lo, hi):
    lo, hi = _add64(lo, hi, _C1_LO, _C1_HI)
    slo, shi = _shr64(lo, hi, 30)
    lo, hi = lo ^ slo, hi ^ shi
    lo, hi = _mul64(lo, hi, _C2_LO, _C2_HI)
    slo, shi = _shr64(lo, hi, 27)
    lo, hi = lo ^ slo, hi ^ shi
    lo, hi = _mul64(lo, hi, _C3_LO, _C3_HI)
    slo, shi = _shr64(lo, hi, 31)
    return lo ^ slo, hi ^ shi


def _hash_body(xb_ref, rlo_ref, rhi_ref, hlo_ref, hhi_ref,
               l0, l1, l2, l3, l4, l5):
    b = xb_ref[...]
    rlo = rlo_ref[...]
    rhi = rhi_ref[...]
    tlo = b * rlo
    thi = _mulhi32(b, rlo) + b * rhi
    def _usum(v):
        s = jnp.sum(lax.bitcast_convert_type(v, jnp.int32), axis=1,
                    dtype=jnp.int32)
        return lax.bitcast_convert_type(s, jnp.uint32)

    sl = _usum(tlo & _M16)
    sh = _usum(tlo >> 16)
    acc_lo = sl + (sh << 16)
    carry = (sh + (sl >> 16)) >> 16
    acc_hi = _usum(thi) + carry
    hlo, hhi = _mix64(acc_lo, acc_hi)
    hlo_ref[...] = hlo
    hhi_ref[...] = hhi
    glo, ghi = _mix64(hlo ^ _CX_LO, hhi ^ _CX_HI)
    l0[...] = (glo & _M12).astype(jnp.int32)
    l1[...] = ((glo >> 12) & _M12).astype(jnp.int32)
    l2[...] = (((glo >> 24) | (ghi << 8)) & _M12).astype(jnp.int32)
    l3[...] = ((ghi >> 4) & _M12).astype(jnp.int32)
    l4[...] = ((ghi >> 16) & _M12).astype(jnp.int32)
    l5[...] = (ghi >> 28).astype(jnp.int32)


def _run_hash(x_bits_pad):
    grid = (_N_PAD // _ROW_BLK,)
    vec_spec = pl.BlockSpec((_ROW_BLK,), lambda i: (i,))
    outs = [jax.ShapeDtypeStruct((_N_PAD,), jnp.uint32)] * 2 + \
           [jax.ShapeDtypeStruct((_N_PAD,), jnp.int32)] * 6
    return pl.pallas_call(
        _hash_body,
        grid=grid,
        in_specs=[
            pl.BlockSpec((_ROW_BLK, _D), lambda i: (i, 0)),
            pl.BlockSpec((1, _D), lambda i: (0, 0)),
            pl.BlockSpec((1, _D), lambda i: (0, 0)),
        ],
        out_specs=[vec_spec] * 8,
        out_shape=outs,
    )(x_bits_pad, _R_LO, _R_HI)


def _scatter_body(src_hbm, dst_hbm, glimbs_hbm, zeros_hbm, out_hbm,
                  src_v, dst_v, b0, b1, b2, b3, b4, b5, b6, b7,
                  acc, gsem, ssem):
    bufs = (b0, b1, b2, b3, b4, b5, b6, b7)
    cid = lax.axis_index("c")
    sid = lax.axis_index("s")
    wid = cid * 16 + sid
    pltpu.sync_copy(src_hbm.at[wid], src_v)
    pltpu.sync_copy(dst_hbm.at[wid], dst_v)

    @pl.when(sid == 0)
    def _():
        pltpu.sync_copy(zeros_hbm, acc)

    plsc.subcore_barrier()

    def g_start(j, b):
        pltpu.async_copy(glimbs_hbm.at[src_v.at[j]], bufs[b], gsem)

    def g_wait(j, b):
        pltpu.make_async_copy(glimbs_hbm.at[src_v.at[j]], bufs[b], gsem).wait()

    def s_start(j, b):
        pltpu.async_copy(bufs[b], acc.at[dst_v.at[j]], ssem, add=True)

    def s_wait(j, b):
        pltpu.make_async_copy(bufs[b], acc.at[dst_v.at[j]], ssem).wait()

    for k in range(4):
        g_start(jnp.int32(k), k)

    def ring(t, carry):
        j0 = t * 8
        for k in range(8):
            j = j0 + k
            g_wait(j, k)
            s_start(j, k)

            @pl.when(j >= 4)
            def _():
                s_wait(j - 4, (k + 4) % 8)

            @pl.when(j + 4 < _NCHUNK)
            def _():
                g_start(j + 4, (k + 4) % 8)
        return carry

    lax.fori_loop(jnp.int32(0), jnp.int32(_NCHUNK // 8), ring, jnp.int32(0))
    for k in range(4):
        jj = jnp.int32(_NCHUNK - 4 + k)
        s_wait(jj, (_NCHUNK - 4 + k) % 8)
    plsc.subcore_barrier()

    @pl.when(sid == 0)
    def _():
        pltpu.sync_copy(acc, out_hbm.at[cid])


@functools.cache
def _make_scatter():
    return functools.partial(
        pl.kernel,
        out_type=jax.ShapeDtypeStruct((2, _N_PAD, _GW), jnp.int32),
        mesh=plsc.VectorSubcoreMesh(core_axis_name="c", subcore_axis_name="s"),
        scratch_types=[
            pltpu.VMEM((_NCHUNK, _CHUNK), jnp.int32),
            pltpu.VMEM((_NCHUNK, _CHUNK), jnp.int32),
            *([pltpu.VMEM((_CHUNK, _GW), jnp.int32)] * 8),
            pltpu.VMEM_SHARED((_N_PAD, _GW), jnp.int32),
            pltpu.SemaphoreType.DMA,
            pltpu.SemaphoreType.DMA,
        ],
        compiler_params=pltpu.CompilerParams(use_tc_tiling_on_sc=False),
    )(_scatter_body)


def _run_scatter(src, dst, glimbs, zeros):
    return _make_scatter()(src, dst, glimbs, zeros)


def _combine_body(hlo_ref, hhi_ref, pa_ref, pb_ref, valid_ref,
                  siglo_ref, sighi_ref):
    s = pa_ref[...] + pb_ref[...]
    s0 = s[0, :]
    s1 = s[1, :]
    s2 = s[2, :]
    s3 = s[3, :]
    s4 = s[4, :]
    s5 = s[5, :]
    lo, hi = s0, jnp.zeros_like(s0)
    lo, hi = _add64(lo, hi, s1 << 12, s1 >> 20)
    lo, hi = _add64(lo, hi, s2 << 24, s2 >> 8)
    hi = hi + (s3 << 4) + (s4 << 16) + (s5 << 28)
    hlo = hlo_ref[...]
    hhi = hhi_ref[...]
    tlo, thi = _mul64(hlo, hhi, _CK_LO, _CK_HI)
    tlo, thi = _add64(tlo, thi, lo, hi)
    siglo, sighi = _mix64(tlo, thi)
    vb = valid_ref[...] != 0
    siglo_ref[...] = jnp.where(vb, siglo, np.uint32(0xFFFFFFFF))
    sighi_ref[...] = jnp.where(vb, sighi, np.uint32(0xFFFFFFFF))


def _run_combine(hlo, hhi, p0, p1, valid):
    grid = (_N_PAD // _ROW_BLK,)
    vec_spec = pl.BlockSpec((_ROW_BLK,), lambda i: (i,))
    tab_spec = pl.BlockSpec((_GW, _ROW_BLK), lambda i: (0, i))
    return pl.pallas_call(
        _combine_body,
        grid=grid,
        in_specs=[vec_spec, vec_spec, tab_spec, tab_spec, vec_spec],
        out_specs=[vec_spec, vec_spec],
        out_shape=[jax.ShapeDtypeStruct((_N_PAD,), jnp.uint32)] * 2,
    )(hlo, hhi, p0, p1, valid)


_FB_ROWS = 40
_FB = _FB_ROWS * _LANE
_FNB = _N_PAD // _FB
_UMAX = np.uint32(0xFFFFFFFF)


def _first_body(jlo_ref, jhi_ref, slo_ref, shi_ref, mlo_ref, mhi_ref):
    jb = pl.program_id(0)
    jl = jlo_ref[...]
    jh = jhi_ref[...]
    jg = jb * _FB + lax.broadcasted_iota(jnp.int32, (_FB_ROWS, _LANE), 0) * _LANE \
        + lax.broadcasted_iota(jnp.int32, (_FB_ROWS, _LANE), 1)

    def body(k, acc):
        kl = slo_ref[0, k]
        kh = shi_ref[0, k]
        m = (jh == kh) & (jl == kl) & (k < jg)
        return acc + m.astype(jnp.int32)

    zero = jnp.zeros((_FB_ROWS, _LANE), jnp.int32)
    acc = lax.switch(jb, [
        lambda u=u: lax.fori_loop(0, (u + 1) * _FB, body, zero, unroll=8)
        for u in range(_FNB)
    ])
    first = acc == 0
    mlo_ref[...] = jnp.where(first, jl, _UMAX)
    mhi_ref[...] = jnp.where(first, jh, _UMAX)


def _run_first(sig2_lo, sig2_hi, sflat_lo, sflat_hi):
    blk = pl.BlockSpec((_FB_ROWS, _LANE), lambda b: (b, 0))
    smem = pl.BlockSpec(memory_space=pltpu.SMEM)
    return pl.pallas_call(
        _first_body,
        grid=(_FNB,),
        in_specs=[blk, blk, smem, smem],
        out_specs=[blk, blk],
        out_shape=[jax.ShapeDtypeStruct((_JB, _LANE), jnp.uint32)] * 2,
    )(sig2_lo, sig2_hi, sflat_lo, sflat_hi)


def _rank_body(ilo_ref, ihi_ref, slo_ref, shi_ref, inv_ref):
    il = ilo_ref[...]
    ih = ihi_ref[...]

    def body(j, acc):
        jl = slo_ref[0, j]
        jh = shi_ref[0, j]
        m = (jh < ih) | ((jh == ih) & (jl < il))
        return acc + m.astype(jnp.int32)

    acc = lax.fori_loop(0, _N_PAD, body,
                        jnp.zeros((_JB, _LANE), jnp.int32), unroll=16)
    inv_ref[...] = acc


def _run_rank(sig2_lo, sig2_hi, mflat_lo, mflat_hi):
    blk = pl.BlockSpec((_JB, _LANE), lambda: (0, 0))
    smem = pl.BlockSpec(memory_space=pltpu.SMEM)
    return pl.pallas_call(
        _rank_body,
        in_specs=[blk, blk, smem, smem],
        out_specs=blk,
        out_shape=jax.ShapeDtypeStruct((_JB, _LANE), jnp.int32),
    )(sig2_lo, sig2_hi, mflat_lo, mflat_hi)


def kernel(x, edge_index):
    with _jax_config.enable_x64(False):
        inv32 = _kernel_impl(x, edge_index)
    return inv32.astype(jnp.int64)


def _kernel_impl(x, edge_index):
    x_bits = lax.bitcast_convert_type(x, jnp.uint32)
    x_bits = jnp.concatenate(
        [x_bits, jnp.zeros((_N_PAD - _N, _D), jnp.uint32)], axis=0)

    hlo, hhi, l0, l1, l2, l3, l4, l5 = _run_hash(x_bits)

    pad_cols = jnp.zeros((_N_PAD, _GW - 6), jnp.int32)
    glimbs = jnp.concatenate(
        [jnp.stack([l0, l1, l2, l3, l4, l5], axis=1), pad_cols], axis=1)

    ei = edge_index.astype(jnp.int32)
    pad_e = jnp.full((_E_PAD - _E,), _TRASH, jnp.int32)
    src = jnp.concatenate([ei[0], pad_e]).reshape(_NUM_TILES, _NCHUNK, _CHUNK)
    dst = jnp.concatenate([ei[1], pad_e]).reshape(_NUM_TILES, _NCHUNK, _CHUNK)

    zeros = jnp.zeros((_N_PAD, _GW), jnp.int32)
    partials = _run_scatter(src, dst, glimbs, zeros)
    p0 = lax.bitcast_convert_type(partials[0].T, jnp.uint32)
    p1 = lax.bitcast_convert_type(partials[1].T, jnp.uint32)

    valid = (jnp.arange(_N_PAD) < _N).astype(jnp.uint32)
    siglo, sighi = _run_combine(hlo, hhi, p0, p1, valid)

    sig2_lo = siglo.reshape(_JB, _LANE)
    sig2_hi = sighi.reshape(_JB, _LANE)
    sflat_lo = siglo.reshape(1, _N_PAD)
    sflat_hi = sighi.reshape(1, _N_PAD)

    mlo, mhi = _run_first(sig2_lo, sig2_hi, sflat_lo, sflat_hi)
    inv = _run_rank(sig2_lo, sig2_hi,
                    mlo.reshape(1, _N_PAD), mhi.reshape(1, _N_PAD))

    return inv.reshape(-1)[:_N]
